# Initial kernel scaffold; baseline (speedup 1.0000x reference)
import jax
import jax.numpy as jnp
from jax.experimental import pallas as pl


def kernel(ids, feats, adjs, edge_emb, Wself0, Wneigh0, Wself1, Wneigh1, Wedge0, Wfc, bfc):
    raise NotImplementedError("write your pallas kernel here")


# speedup vs baseline: 1.7043x; 1.7043x over previous
import functools

import jax
import jax.numpy as jnp
from jax import lax
from jax.experimental import pallas as pl
from jax.experimental.pallas import tpu as pltpu
from jax.experimental.pallas import tpu_sc as plsc

N_NODES = 50000
MAX_DEG = 16
D_IN = 128
D_EDGE = 16
S = 10
B = 1024

NC, NS, L = 2, 16, 16
NW = NC * NS
CHUNK = B // NW
E2 = CHUNK * S
GCH = 64
NG = E2 // GCH


def _sc_body(ids_h, feats_h, adjc_h, eec_h,
             f0_o, f1_o, e0_o, e1s_o, f2s_o,
             ids_v, idsm_v, adjr_v, cur1a_v, cur1_v, cur1m_v, adj1r_v,
             e0idx_v, iv_v, kv_v,
             f2idx_v, e1idx_v,
             f0_v, f1_v, e0_v, e1rows_v, e1sum_v, f2rows_v, f2sum_v,
             sem0, sem1, sem2, sem3, sem4):
    wid = lax.axis_index("s") * NC + lax.axis_index("c")
    base = wid * CHUNK

    pltpu.sync_copy(ids_h.at[pl.ds(base, CHUNK)], ids_v)

    pltpu.async_copy(feats_h.at[ids_v], f0_v, sem0).wait()
    pltpu.sync_copy(f0_v, f0_o.at[pl.ds(base, CHUNK), :])

    for t in range(E2 // L):
        pvec = lax.broadcasted_iota(jnp.int32, (L,), 0) + t * L
        ivec = pvec // S
        kvec = pvec - ivec * S
        iv_v[pl.ds(t * L, L)] = ivec
        kv_v[pl.ds(t * L, L)] = kvec

    for mp in range(2):
        mpn = mp * N_NODES
        mpe = mp * N_NODES * MAX_DEG

        for j in range(S):
            for h in range(CHUNK // L):
                e0idx_v[j, pl.ds(h * L, L)] = (
                    ids_v[pl.ds(h * L, L)] * MAX_DEG + (mpe + j))
        for h in range(CHUNK // L):
            idsm_v[pl.ds(h * L, L)] = ids_v[pl.ds(h * L, L)] + mpn

        pltpu.async_copy(adjc_h.at[idsm_v], adjr_v, sem0).wait()

        for j in range(S):
            for h in range(CHUNK // L):
                lanes = lax.broadcasted_iota(jnp.int32, (L,), 0) + h * L
                c1 = plsc.load_gather(
                    adjr_v, [lanes, jnp.full((L,), j, jnp.int32)])
                cur1a_v[j, pl.ds(h * L, L)] = c1

        @pl.loop(0, S)
        def _per_neighbor(j):
            for h in range(CHUNK // L):
                cvec = cur1a_v[j, pl.ds(h * L, L)]
                cur1_v[pl.ds(h * L, L)] = cvec
                cur1m_v[pl.ds(h * L, L)] = cvec + mpn

            cp_e0 = pltpu.async_copy(eec_h.at[e0idx_v.at[j]], e0_v, sem1)
            cp_f1 = pltpu.async_copy(feats_h.at[cur1_v], f1_v, sem2)
            cp_a1 = pltpu.async_copy(adjc_h.at[cur1m_v], adj1r_v, sem3)

            for t in range(E2 // L):
                ivec = iv_v[pl.ds(t * L, L)]
                kvec = kv_v[pl.ds(t * L, L)]
                c = plsc.load_gather(cur1_v, [ivec])
                e1idx_v[t // 4, pl.ds((t % 4) * L, L)] = (
                    c * MAX_DEG + (kvec + mpe))
            e1_cps = [
                pltpu.async_copy(eec_h.at[e1idx_v.at[u]],
                                 e1rows_v.at[pl.ds(u * GCH, GCH), :], sem4)
                for u in range(NG)
            ]

            cp_a1.wait()
            for t in range(E2 // L):
                ivec = iv_v[pl.ds(t * L, L)]
                kvec = kv_v[pl.ds(t * L, L)]
                a = plsc.load_gather(adj1r_v, [ivec, kvec])
                f2idx_v[t // 4, pl.ds((t % 4) * L, L)] = a
            f2_cps = [
                pltpu.async_copy(feats_h.at[f2idx_v.at[u]],
                                 f2rows_v.at[pl.ds(u * GCH, GCH), :], sem0)
                for u in range(NG)
            ]

            cp_e0.wait()
            pltpu.sync_copy(e0_v, e0_o.at[mp, j, pl.ds(base, CHUNK), :])
            cp_f1.wait()
            pltpu.sync_copy(f1_v, f1_o.at[mp, j, pl.ds(base, CHUNK), :])

            for cp in e1_cps:
                cp.wait()

            @pl.loop(0, CHUNK)
            def _e1sum(i):
                r0 = i * S
                acc = e1rows_v[r0, :]
                for k in range(1, S):
                    acc = acc + e1rows_v[r0 + k, :]
                e1sum_v[i, :] = acc

            for cp in f2_cps:
                cp.wait()

            @pl.loop(0, CHUNK)
            def _f2sum(i):
                r0 = i * S
                for c in range(D_IN // L):
                    acc = f2rows_v[r0, pl.ds(c * L, L)]
                    for k in range(1, S):
                        acc = acc + f2rows_v[r0 + k, pl.ds(c * L, L)]
                    f2sum_v[i, pl.ds(c * L, L)] = acc

            pltpu.sync_copy(e1sum_v, e1s_o.at[mp, j, pl.ds(base, CHUNK), :])
            pltpu.sync_copy(f2sum_v, f2s_o.at[mp, j, pl.ds(base, CHUNK), :])


_sc_gather = pl.kernel(
    _sc_body,
    out_type=[
        jax.ShapeDtypeStruct((B, D_IN), jnp.float32),
        jax.ShapeDtypeStruct((2, S, B, D_IN), jnp.float32),
        jax.ShapeDtypeStruct((2, S, B, D_EDGE), jnp.float32),
        jax.ShapeDtypeStruct((2, S, B, D_EDGE), jnp.float32),
        jax.ShapeDtypeStruct((2, S, B, D_IN), jnp.float32),
    ],
    mesh=plsc.VectorSubcoreMesh(core_axis_name="c", subcore_axis_name="s",
                                num_cores=NC, num_subcores=NS),
    compiler_params=pltpu.CompilerParams(needs_layout_passes=False,
                                         use_tc_tiling_on_sc=False),
    scratch_types=[
        pltpu.VMEM((CHUNK,), jnp.int32),
        pltpu.VMEM((CHUNK,), jnp.int32),
        pltpu.VMEM((CHUNK, MAX_DEG), jnp.int32),
        pltpu.VMEM((S, CHUNK), jnp.int32),
        pltpu.VMEM((CHUNK,), jnp.int32),
        pltpu.VMEM((CHUNK,), jnp.int32),
        pltpu.VMEM((CHUNK, MAX_DEG), jnp.int32),
        pltpu.VMEM((S, CHUNK), jnp.int32),
        pltpu.VMEM((E2,), jnp.int32),
        pltpu.VMEM((E2,), jnp.int32),
        pltpu.VMEM((NG, GCH), jnp.int32),
        pltpu.VMEM((NG, GCH), jnp.int32),
        pltpu.VMEM((CHUNK, D_IN), jnp.float32),
        pltpu.VMEM((CHUNK, D_IN), jnp.float32),
        pltpu.VMEM((CHUNK, D_EDGE), jnp.float32),
        pltpu.VMEM((E2, D_EDGE), jnp.float32),
        pltpu.VMEM((CHUNK, D_EDGE), jnp.float32),
        pltpu.VMEM((E2, D_IN), jnp.float32),
        pltpu.VMEM((CHUNK, D_IN), jnp.float32),
        pltpu.SemaphoreType.DMA,
        pltpu.SemaphoreType.DMA,
        pltpu.SemaphoreType.DMA,
        pltpu.SemaphoreType.DMA,
        pltpu.SemaphoreType.DMA,
    ],
)


def _tr_body(x_ref, o_ref):
    x = x_ref[0]
    o_ref[0] = x.T


def _make_transpose(dtype, minor, ec):
    return pl.pallas_call(
        _tr_body,
        grid=(2, minor // ec),
        in_specs=[pl.BlockSpec((1, 16, ec), lambda mp, t: (mp, 0, t))],
        out_specs=pl.BlockSpec((1, ec, 16), lambda mp, t: (mp, t, 0)),
        out_shape=jax.ShapeDtypeStruct((2, minor, 16), dtype),
        compiler_params=pltpu.CompilerParams(
            dimension_semantics=("parallel", "parallel"),
        ),
    )


_tr_ee = _make_transpose(jnp.float32, N_NODES * MAX_DEG, 16000)
_tr_adj = _make_transpose(jnp.int32, N_NODES, N_NODES)


def _relu(x):
    return jnp.maximum(x, 0.0)


def _mm(a, b):
    return jnp.dot(a, b, preferred_element_type=jnp.float32)


def _tc_body(f0_r, f1_r, e0_r, e1s_r, f2s_r,
             ws0_r, wn0_r, ws1_r, wn1_r, we_r, wfc_r, bfc_r,
             out_r, acc_r, f1s_v, e0s_v, g1s_v, e0ps_v):
    mp = pl.program_id(0)
    inv_s = 1.0 / S

    ws0 = ws0_r[0]
    wn0a = wn0_r[0, :D_IN, :]
    wn0b = wn0_r[0, D_IN:, :]
    ws1 = ws1_r[0]
    wn1a = wn1_r[0, :256, :]
    wn1b = wn1_r[0, 256:, :]
    wea = we_r[0, :256, :]
    web = we_r[0, 256:512, :]
    wec = we_r[0, 512:, :]

    f1s_v[...] = jnp.zeros_like(f1s_v)
    e0s_v[...] = jnp.zeros_like(e0s_v)

    def body1(j, _):
        f1s_v[...] += f1_r[0, j]
        e0s_v[...] += e0_r[0, j]
        return 0

    lax.fori_loop(0, S, body1, 0)

    f0 = f0_r[...]
    g0 = _relu(jnp.concatenate(
        [_mm(f0, ws0),
         _mm(f1s_v[...] * inv_s, wn0a) + _mm(e0s_v[...] * inv_s, wn0b)],
        axis=-1))
    g0ea = _mm(g0, wea)

    g1s_v[...] = jnp.zeros_like(g1s_v)
    e0ps_v[...] = jnp.zeros_like(e0ps_v)

    def body2(j, _):
        g1_j = _relu(jnp.concatenate(
            [_mm(f1_r[0, j], ws0),
             _mm(f2s_r[0, j] * inv_s, wn0a) + _mm(e1s_r[0, j] * inv_s, wn0b)],
            axis=-1))
        e0p_j = _relu(g0ea + _mm(g1_j, web) + _mm(e0_r[0, j], wec))
        g1s_v[...] += g1_j
        e0ps_v[...] += e0p_j
        return 0

    lax.fori_loop(0, S, body2, 0)

    out_mp = _relu(jnp.concatenate(
        [_mm(g0, ws1),
         _mm(g1s_v[...] * inv_s, wn1a) + _mm(e0ps_v[...] * inv_s, wn1b)],
        axis=-1))

    @pl.when(mp == 0)
    def _():
        acc_r[...] = out_mp

    @pl.when(mp == 1)
    def _():
        o = (acc_r[...] + out_mp) * 0.5
        nrm = jnp.sqrt(jnp.sum(o * o, axis=1, keepdims=True))
        o = o / jnp.maximum(nrm, 1e-12)
        out_r[...] = _mm(o, wfc_r[...]) + bfc_r[...]


def _tc_dense(f0, f1, e0, e1s, f2s, Wself0, Wneigh0, Wself1, Wneigh1,
              Wedge0, Wfc, bfc2):
    grid = (2,)
    return pl.pallas_call(
        _tc_body,
        grid=grid,
        in_specs=[
            pl.BlockSpec((B, D_IN), lambda mp: (0, 0)),
            pl.BlockSpec((1, S, B, D_IN), lambda mp: (mp, 0, 0, 0)),
            pl.BlockSpec((1, S, B, D_EDGE), lambda mp: (mp, 0, 0, 0)),
            pl.BlockSpec((1, S, B, D_EDGE), lambda mp: (mp, 0, 0, 0)),
            pl.BlockSpec((1, S, B, D_IN), lambda mp: (mp, 0, 0, 0)),
            pl.BlockSpec((1, D_IN, 128), lambda mp: (mp, 0, 0)),
            pl.BlockSpec((1, D_IN + D_EDGE, 128), lambda mp: (mp, 0, 0)),
            pl.BlockSpec((1, 256, 128), lambda mp: (mp, 0, 0)),
            pl.BlockSpec((1, 256 + D_EDGE, 128), lambda mp: (mp, 0, 0)),
            pl.BlockSpec((1, 512 + D_EDGE, D_EDGE), lambda mp: (mp, 0, 0)),
            pl.BlockSpec((256, 16), lambda mp: (0, 0)),
            pl.BlockSpec((1, 16), lambda mp: (0, 0)),
        ],
        out_specs=pl.BlockSpec((B, 16), lambda mp: (0, 0)),
        out_shape=jax.ShapeDtypeStruct((B, 16), jnp.float32),
        scratch_shapes=[
            pltpu.VMEM((B, 256), jnp.float32),
            pltpu.VMEM((B, D_IN), jnp.float32),
            pltpu.VMEM((B, D_EDGE), jnp.float32),
            pltpu.VMEM((B, 256), jnp.float32),
            pltpu.VMEM((B, D_EDGE), jnp.float32),
        ],
        compiler_params=pltpu.CompilerParams(
            dimension_semantics=("arbitrary",),
        ),
    )(f0, f1, e0, e1s, f2s, Wself0, Wneigh0, Wself1, Wneigh1, Wedge0,
      Wfc, bfc2)


def kernel(ids, feats, adjs, edge_emb, Wself0, Wneigh0, Wself1, Wneigh1,
           Wedge0, Wfc, bfc):
    ee_c = _tr_ee(jnp.transpose(edge_emb, (0, 2, 1)))
    ad_c = _tr_adj(jnp.transpose(adjs, (0, 2, 1)))
    ee_flat = ee_c.reshape(2 * N_NODES * MAX_DEG, D_EDGE)
    ad_flat = ad_c.reshape(2 * N_NODES, MAX_DEG)
    f0, f1, e0, e1s, f2s = _sc_gather(ids, feats, ad_flat, ee_flat)
    return _tc_dense(f0, f1, e0, e1s, f2s, Wself0, Wneigh0, Wself1,
                     Wneigh1, Wedge0, Wfc, bfc.reshape(1, 16))


# speedup vs baseline: 6.1631x; 3.6161x over previous
import functools

import jax
import jax.numpy as jnp
from jax import lax
from jax.experimental import pallas as pl
from jax.experimental.pallas import tpu as pltpu
from jax.experimental.pallas import tpu_sc as plsc

N_NODES = 50000
MAX_DEG = 16
D_IN = 128
D_EDGE = 16
S = 10
B = 1024

NC, NS, L = 2, 16, 16
NW = NC * NS
CHUNK = B // NW
E2 = CHUNK * S
GCH = 64
NG = E2 // GCH


def _sc_body(ids_h, feats_h, adjc_h, eeb_h,
             f0_o, f1_o, e0_o, e1s_o, f2s_o,
             ids_v, idsm_v, adjr_v, cur1a_v, cur1_v, cur1m_v, adj1r_v,
             nb_v, ebidx_v, erows_v, e0all_v, iv_v, kv_v,
             f2idx_v,
             f0_v, f1_v, e1sum_v, f2rows_v, f2sum_v,
             sem0, sem1, sem2, sem3, sem4):
    wid = lax.axis_index("s") * NC + lax.axis_index("c")
    base = wid * CHUNK

    pltpu.sync_copy(ids_h.at[pl.ds(base, CHUNK)], ids_v)

    pltpu.async_copy(feats_h.at[ids_v], f0_v, sem0).wait()
    pltpu.sync_copy(f0_v, f0_o.at[pl.ds(base, CHUNK), :])

    for t in range(E2 // L):
        pvec = lax.broadcasted_iota(jnp.int32, (L,), 0) + t * L
        ivec = pvec // S
        kvec = pvec - ivec * S
        iv_v[pl.ds(t * L, L)] = ivec
        kv_v[pl.ds(t * L, L)] = kvec

    lanes0 = lax.broadcasted_iota(jnp.int32, (L,), 0)

    for mp in range(2):
        mpn = mp * N_NODES

        for h in range(CHUNK // L):
            idsm_v[pl.ds(h * L, L)] = ids_v[pl.ds(h * L, L)] + mpn

        cp_adj = pltpu.async_copy(adjc_h.at[idsm_v], adjr_v, sem0)

        for h in range(CHUNK // L):
            v = ids_v[pl.ds(h * L, L)]
            nb_v[pl.ds(h * L, L)] = (v >> 3) * 64 + (v & 7)
        for d in range(D_EDGE):
            rc = mp * 800000 + (d // 8) * 400000 + (d % 8) * 8
            for h in range(CHUNK // L):
                c = d * 2 + h
                ebidx_v[c // 8, pl.ds((c % 8) * L, L)] = (
                    nb_v[pl.ds(h * L, L)] + rc)
        e0_cps = [pltpu.async_copy(eeb_h.at[ebidx_v.at[u]],
                                   erows_v.at[pl.ds(u * 128, 128), :], sem1)
                  for u in range(4)]
        for cp in e0_cps:
            cp.wait()
        for j in range(S):
            for d in range(D_EDGE):
                for h in range(CHUNK // L):
                    c = d * 2 + h
                    v = plsc.load_gather(
                        erows_v,
                        [lanes0 + c * L, jnp.full((L,), j, jnp.int32)])
                    plsc.store_scatter(
                        e0all_v.at[j],
                        [lanes0 + h * L, jnp.full((L,), d, jnp.int32)], v)
        for j in range(S):
            pltpu.sync_copy(e0all_v.at[j],
                            e0_o.at[mp, j, pl.ds(base, CHUNK), :])

        cp_adj.wait()

        for j in range(S):
            for h in range(CHUNK // L):
                lanes = lax.broadcasted_iota(jnp.int32, (L,), 0) + h * L
                c1 = plsc.load_gather(
                    adjr_v, [lanes, jnp.full((L,), j, jnp.int32)])
                cur1a_v[j, pl.ds(h * L, L)] = c1

        @pl.loop(0, S)
        def _per_neighbor(j):
            for h in range(CHUNK // L):
                cvec = cur1a_v[j, pl.ds(h * L, L)]
                cur1_v[pl.ds(h * L, L)] = cvec
                cur1m_v[pl.ds(h * L, L)] = cvec + mpn

            cp_f1 = pltpu.async_copy(feats_h.at[cur1_v], f1_v, sem2)
            cp_a1 = pltpu.async_copy(adjc_h.at[cur1m_v], adj1r_v, sem3)

            for h in range(CHUNK // L):
                v = cur1_v[pl.ds(h * L, L)]
                nb_v[pl.ds(h * L, L)] = (v >> 3) * 64 + (v & 7)
            for d in range(D_EDGE):
                rc = mp * 800000 + (d // 8) * 400000 + (d % 8) * 8
                for h in range(CHUNK // L):
                    c = d * 2 + h
                    ebidx_v[c // 8, pl.ds((c % 8) * L, L)] = (
                        nb_v[pl.ds(h * L, L)] + rc)
            e1_cps = [
                pltpu.async_copy(eeb_h.at[ebidx_v.at[u]],
                                 erows_v.at[pl.ds(u * 128, 128), :], sem4)
                for u in range(4)
            ]

            cp_a1.wait()
            for t in range(E2 // L):
                ivec = iv_v[pl.ds(t * L, L)]
                kvec = kv_v[pl.ds(t * L, L)]
                a = plsc.load_gather(adj1r_v, [ivec, kvec])
                f2idx_v[t // 4, pl.ds((t % 4) * L, L)] = a
            f2_cps = [
                pltpu.async_copy(feats_h.at[f2idx_v.at[u]],
                                 f2rows_v.at[pl.ds(u * GCH, GCH), :], sem0)
                for u in range(NG)
            ]

            cp_f1.wait()
            pltpu.sync_copy(f1_v, f1_o.at[mp, j, pl.ds(base, CHUNK), :])

            for cp in e1_cps:
                cp.wait()

            for d in range(D_EDGE):
                for h in range(CHUNK // L):
                    c = d * 2 + h
                    acc = plsc.load_gather(
                        erows_v,
                        [lanes0 + c * L, jnp.full((L,), 0, jnp.int32)])
                    for k in range(1, S):
                        acc = acc + plsc.load_gather(
                            erows_v,
                            [lanes0 + c * L, jnp.full((L,), k, jnp.int32)])
                    plsc.store_scatter(
                        e1sum_v,
                        [lanes0 + h * L, jnp.full((L,), d, jnp.int32)], acc)

            for cp in f2_cps:
                cp.wait()

            @pl.loop(0, CHUNK)
            def _f2sum(i):
                r0 = i * S
                for c in range(D_IN // L):
                    acc = f2rows_v[r0, pl.ds(c * L, L)]
                    for k in range(1, S):
                        acc = acc + f2rows_v[r0 + k, pl.ds(c * L, L)]
                    f2sum_v[i, pl.ds(c * L, L)] = acc

            pltpu.sync_copy(e1sum_v, e1s_o.at[mp, j, pl.ds(base, CHUNK), :])
            pltpu.sync_copy(f2sum_v, f2s_o.at[mp, j, pl.ds(base, CHUNK), :])


_sc_gather = pl.kernel(
    _sc_body,
    out_type=[
        jax.ShapeDtypeStruct((B, D_IN), jnp.float32),
        jax.ShapeDtypeStruct((2, S, B, D_IN), jnp.float32),
        jax.ShapeDtypeStruct((2, S, B, D_EDGE), jnp.float32),
        jax.ShapeDtypeStruct((2, S, B, D_EDGE), jnp.float32),
        jax.ShapeDtypeStruct((2, S, B, D_IN), jnp.float32),
    ],
    mesh=plsc.VectorSubcoreMesh(core_axis_name="c", subcore_axis_name="s",
                                num_cores=NC, num_subcores=NS),
    compiler_params=pltpu.CompilerParams(needs_layout_passes=False,
                                         use_tc_tiling_on_sc=False),
    scratch_types=[
        pltpu.VMEM((CHUNK,), jnp.int32),
        pltpu.VMEM((CHUNK,), jnp.int32),
        pltpu.VMEM((CHUNK, MAX_DEG), jnp.int32),
        pltpu.VMEM((S, CHUNK), jnp.int32),
        pltpu.VMEM((CHUNK,), jnp.int32),
        pltpu.VMEM((CHUNK,), jnp.int32),
        pltpu.VMEM((CHUNK, MAX_DEG), jnp.int32),
        pltpu.VMEM((CHUNK,), jnp.int32),
        pltpu.VMEM((4, 128), jnp.int32),
        pltpu.VMEM((512, D_EDGE), jnp.float32),
        pltpu.VMEM((S, CHUNK, D_EDGE), jnp.float32),
        pltpu.VMEM((E2,), jnp.int32),
        pltpu.VMEM((E2,), jnp.int32),
        pltpu.VMEM((NG, GCH), jnp.int32),
        pltpu.VMEM((CHUNK, D_IN), jnp.float32),
        pltpu.VMEM((CHUNK, D_IN), jnp.float32),
        pltpu.VMEM((CHUNK, D_EDGE), jnp.float32),
        pltpu.VMEM((E2, D_IN), jnp.float32),
        pltpu.VMEM((CHUNK, D_IN), jnp.float32),
        pltpu.SemaphoreType.DMA,
        pltpu.SemaphoreType.DMA,
        pltpu.SemaphoreType.DMA,
        pltpu.SemaphoreType.DMA,
        pltpu.SemaphoreType.DMA,
    ],
)


def _tr_body(x_ref, o_ref):
    x = x_ref[0]
    o_ref[0] = x.T


def _make_transpose(dtype, minor, ec):
    return pl.pallas_call(
        _tr_body,
        grid=(2, minor // ec),
        in_specs=[pl.BlockSpec((1, 16, ec), lambda mp, t: (mp, 0, t))],
        out_specs=pl.BlockSpec((1, ec, 16), lambda mp, t: (mp, t, 0)),
        out_shape=jax.ShapeDtypeStruct((2, minor, 16), dtype),
        compiler_params=pltpu.CompilerParams(
            dimension_semantics=("parallel", "parallel"),
        ),
    )


_tr_adj = _make_transpose(jnp.int32, N_NODES, N_NODES)


def _relu(x):
    return jnp.maximum(x, 0.0)


def _mm(a, b):
    return jnp.dot(a, b, preferred_element_type=jnp.float32)


def _tc_body(f0_r, f1_r, e0_r, e1s_r, f2s_r,
             ws0_r, wn0_r, ws1_r, wn1_r, we_r, wfc_r, bfc_r,
             out_r, acc_r, f1s_v, e0s_v, g1s_v, e0ps_v):
    mp = pl.program_id(0)
    inv_s = 1.0 / S

    ws0 = ws0_r[0]
    wn0a = wn0_r[0, :D_IN, :]
    wn0b = wn0_r[0, D_IN:, :]
    ws1 = ws1_r[0]
    wn1a = wn1_r[0, :256, :]
    wn1b = wn1_r[0, 256:, :]
    wea = we_r[0, :256, :]
    web = we_r[0, 256:512, :]
    wec = we_r[0, 512:, :]

    f1s_v[...] = jnp.zeros_like(f1s_v)
    e0s_v[...] = jnp.zeros_like(e0s_v)

    def body1(j, _):
        f1s_v[...] += f1_r[0, j]
        e0s_v[...] += e0_r[0, j]
        return 0

    lax.fori_loop(0, S, body1, 0)

    f0 = f0_r[...]
    g0 = _relu(jnp.concatenate(
        [_mm(f0, ws0),
         _mm(f1s_v[...] * inv_s, wn0a) + _mm(e0s_v[...] * inv_s, wn0b)],
        axis=-1))
    g0ea = _mm(g0, wea)

    g1s_v[...] = jnp.zeros_like(g1s_v)
    e0ps_v[...] = jnp.zeros_like(e0ps_v)

    def body2(j, _):
        g1_j = _relu(jnp.concatenate(
            [_mm(f1_r[0, j], ws0),
             _mm(f2s_r[0, j] * inv_s, wn0a) + _mm(e1s_r[0, j] * inv_s, wn0b)],
            axis=-1))
        e0p_j = _relu(g0ea + _mm(g1_j, web) + _mm(e0_r[0, j], wec))
        g1s_v[...] += g1_j
        e0ps_v[...] += e0p_j
        return 0

    lax.fori_loop(0, S, body2, 0)

    out_mp = _relu(jnp.concatenate(
        [_mm(g0, ws1),
         _mm(g1s_v[...] * inv_s, wn1a) + _mm(e0ps_v[...] * inv_s, wn1b)],
        axis=-1))

    @pl.when(mp == 0)
    def _():
        acc_r[...] = out_mp

    @pl.when(mp == 1)
    def _():
        o = (acc_r[...] + out_mp) * 0.5
        nrm = jnp.sqrt(jnp.sum(o * o, axis=1, keepdims=True))
        o = o / jnp.maximum(nrm, 1e-12)
        out_r[...] = _mm(o, wfc_r[...]) + bfc_r[...]


def _tc_dense(f0, f1, e0, e1s, f2s, Wself0, Wneigh0, Wself1, Wneigh1,
              Wedge0, Wfc, bfc2):
    grid = (2,)
    return pl.pallas_call(
        _tc_body,
        grid=grid,
        in_specs=[
            pl.BlockSpec((B, D_IN), lambda mp: (0, 0)),
            pl.BlockSpec((1, S, B, D_IN), lambda mp: (mp, 0, 0, 0)),
            pl.BlockSpec((1, S, B, D_EDGE), lambda mp: (mp, 0, 0, 0)),
            pl.BlockSpec((1, S, B, D_EDGE), lambda mp: (mp, 0, 0, 0)),
            pl.BlockSpec((1, S, B, D_IN), lambda mp: (mp, 0, 0, 0)),
            pl.BlockSpec((1, D_IN, 128), lambda mp: (mp, 0, 0)),
            pl.BlockSpec((1, D_IN + D_EDGE, 128), lambda mp: (mp, 0, 0)),
            pl.BlockSpec((1, 256, 128), lambda mp: (mp, 0, 0)),
            pl.BlockSpec((1, 256 + D_EDGE, 128), lambda mp: (mp, 0, 0)),
            pl.BlockSpec((1, 512 + D_EDGE, D_EDGE), lambda mp: (mp, 0, 0)),
            pl.BlockSpec((256, 16), lambda mp: (0, 0)),
            pl.BlockSpec((1, 16), lambda mp: (0, 0)),
        ],
        out_specs=pl.BlockSpec((B, 16), lambda mp: (0, 0)),
        out_shape=jax.ShapeDtypeStruct((B, 16), jnp.float32),
        scratch_shapes=[
            pltpu.VMEM((B, 256), jnp.float32),
            pltpu.VMEM((B, D_IN), jnp.float32),
            pltpu.VMEM((B, D_EDGE), jnp.float32),
            pltpu.VMEM((B, 256), jnp.float32),
            pltpu.VMEM((B, D_EDGE), jnp.float32),
        ],
        compiler_params=pltpu.CompilerParams(
            dimension_semantics=("arbitrary",),
        ),
    )(f0, f1, e0, e1s, f2s, Wself0, Wneigh0, Wself1, Wneigh1, Wedge0,
      Wfc, bfc2)


def kernel(ids, feats, adjs, edge_emb, Wself0, Wneigh0, Wself1, Wneigh1,
           Wedge0, Wfc, bfc):
    eeb = jnp.transpose(edge_emb, (0, 2, 1)).reshape(2, 2, 8, 6250, 128)
    eeb = jnp.transpose(eeb, (0, 1, 3, 2, 4)).reshape(
        2 * MAX_DEG * N_NODES, D_EDGE)
    ad_c = _tr_adj(jnp.transpose(adjs, (0, 2, 1)))
    ad_flat = ad_c.reshape(2 * N_NODES, MAX_DEG)
    f0, f1, e0, e1s, f2s = _sc_gather(ids, feats, ad_flat, eeb)
    return _tc_dense(f0, f1, e0, e1s, f2s, Wself0, Wneigh0, Wself1,
                     Wneigh1, Wedge0, Wfc, bfc.reshape(1, 16))


# speedup vs baseline: 6.5137x; 1.0569x over previous
import functools

import jax
import jax.numpy as jnp
from jax import lax
from jax.experimental import pallas as pl
from jax.experimental.pallas import tpu as pltpu
from jax.experimental.pallas import tpu_sc as plsc

N_NODES = 50000
MAX_DEG = 16
D_IN = 128
D_EDGE = 16
S = 10
B = 1024

NC, NS, L = 2, 16, 16
NW = NC * NS
CHUNK = B // NW
E2 = CHUNK * S
GCH = 64
NG = E2 // GCH


def _sc_body(ids_h, feats_h, adjc_h, eeb_h,
             f0_o, f1_o, e0_o, e1s_o, f2s_o,
             ids_v, idsm_v, adjr_v, cur1a_v, nb_v,
             cur1_v0, cur1_v1, cur1m_v0, cur1m_v1,
             adj1r_v0, adj1r_v1, f2idx_v0, f2idx_v1,
             ebidx_v0, ebidx_v1, erows_v0, erows_v1,
             f1_v0, f1_v1, f2rows_v0, f2rows_v1,
             e0all_v, e1sum_v, f2sum_v, iv_v, kv_v,
             sem0, sem_f1_0, sem_f1_1, sem_a1_0, sem_a1_1,
             sem_e1_0, sem_e1_1, sem_f2_0, sem_f2_1):
    wid = lax.axis_index("s") * NC + lax.axis_index("c")
    base = wid * CHUNK

    cur1_v = (cur1_v0, cur1_v1)
    cur1m_v = (cur1m_v0, cur1m_v1)
    adj1r_v = (adj1r_v0, adj1r_v1)
    f2idx_v = (f2idx_v0, f2idx_v1)
    ebidx_v = (ebidx_v0, ebidx_v1)
    erows_v = (erows_v0, erows_v1)
    f1_v = (f1_v0, f1_v1)
    f2rows_v = (f2rows_v0, f2rows_v1)
    sem_f1 = (sem_f1_0, sem_f1_1)
    sem_a1 = (sem_a1_0, sem_a1_1)
    sem_e1 = (sem_e1_0, sem_e1_1)
    sem_f2 = (sem_f2_0, sem_f2_1)

    pltpu.sync_copy(ids_h.at[pl.ds(base, CHUNK)], ids_v)

    pltpu.async_copy(feats_h.at[ids_v], f1_v0, sem0).wait()
    pltpu.sync_copy(f1_v0, f0_o.at[pl.ds(base, CHUNK), :])

    for t in range(E2 // L):
        pvec = lax.broadcasted_iota(jnp.int32, (L,), 0) + t * L
        ivec = pvec // S
        kvec = pvec - ivec * S
        iv_v[pl.ds(t * L, L)] = ivec
        kv_v[pl.ds(t * L, L)] = kvec

    lanes0 = lax.broadcasted_iota(jnp.int32, (L,), 0)

    for mp in range(2):
        mpn = mp * N_NODES

        for h in range(CHUNK // L):
            idsm_v[pl.ds(h * L, L)] = ids_v[pl.ds(h * L, L)] + mpn

        cp_adj = pltpu.async_copy(adjc_h.at[idsm_v], adjr_v, sem0)

        for h in range(CHUNK // L):
            v = ids_v[pl.ds(h * L, L)]
            nb_v[pl.ds(h * L, L)] = (v >> 3) * 64 + (v & 7)
        for d in range(D_EDGE):
            rc = mp * 800000 + (d // 8) * 400000 + (d % 8) * 8
            for h in range(CHUNK // L):
                c = d * 2 + h
                ebidx_v0[c // 8, pl.ds((c % 8) * L, L)] = (
                    nb_v[pl.ds(h * L, L)] + rc)
        e0_cps = [pltpu.async_copy(eeb_h.at[ebidx_v0.at[u]],
                                   erows_v0.at[pl.ds(u * 128, 128), :],
                                   sem_e1_0)
                  for u in range(4)]
        for cp in e0_cps:
            cp.wait()

        @pl.loop(0, S)
        def _e0extract(j):
            for d in range(D_EDGE):
                for h in range(CHUNK // L):
                    c = d * 2 + h
                    v = plsc.load_gather(
                        erows_v0,
                        [lanes0 + c * L, jnp.full((L,), 0, jnp.int32) + j])
                    plsc.store_scatter(
                        e0all_v.at[j],
                        [lanes0 + h * L, jnp.full((L,), d, jnp.int32)], v)
            pltpu.sync_copy(e0all_v.at[j],
                            e0_o.at[mp, j, pl.ds(base, CHUNK), :])

        cp_adj.wait()

        for j in range(S):
            for h in range(CHUNK // L):
                lanes = lax.broadcasted_iota(jnp.int32, (L,), 0) + h * L
                c1 = plsc.load_gather(
                    adjr_v, [lanes, jnp.full((L,), j, jnp.int32)])
                cur1a_v[j, pl.ds(h * L, L)] = c1


        def front(j, b):
            for h in range(CHUNK // L):
                cvec = cur1a_v[j, pl.ds(h * L, L)]
                cur1_v[b][pl.ds(h * L, L)] = cvec
                cur1m_v[b][pl.ds(h * L, L)] = cvec + mpn
                nb_v[pl.ds(h * L, L)] = (cvec >> 3) * 64 + (cvec & 7)
            pltpu.async_copy(feats_h.at[cur1_v[b]], f1_v[b], sem_f1[b])
            pltpu.async_copy(adjc_h.at[cur1m_v[b]], adj1r_v[b], sem_a1[b])
            for d in range(D_EDGE):
                rc = mp * 800000 + (d // 8) * 400000 + (d % 8) * 8
                for h in range(CHUNK // L):
                    c = d * 2 + h
                    ebidx_v[b][c // 8, pl.ds((c % 8) * L, L)] = (
                        nb_v[pl.ds(h * L, L)] + rc)
            for u in range(4):
                pltpu.async_copy(eeb_h.at[ebidx_v[b].at[u]],
                                 erows_v[b].at[pl.ds(u * 128, 128), :],
                                 sem_e1[b])

        def mid(j, b):
            pltpu.make_async_copy(adjc_h.at[cur1m_v[b]], adj1r_v[b],
                                  sem_a1[b]).wait()
            for t in range(E2 // L):
                ivec = iv_v[pl.ds(t * L, L)]
                kvec = kv_v[pl.ds(t * L, L)]
                a = plsc.load_gather(adj1r_v[b], [ivec, kvec])
                f2idx_v[b][t // 4, pl.ds((t % 4) * L, L)] = a
            for u in range(NG):
                pltpu.async_copy(feats_h.at[f2idx_v[b].at[u]],
                                 f2rows_v[b].at[pl.ds(u * GCH, GCH), :],
                                 sem_f2[b])

        def back(j, b):
            pltpu.make_async_copy(feats_h.at[cur1_v[b]], f1_v[b],
                                  sem_f1[b]).wait()
            pltpu.sync_copy(f1_v[b], f1_o.at[mp, j, pl.ds(base, CHUNK), :])
            for u in range(4):
                pltpu.make_async_copy(eeb_h.at[ebidx_v[b].at[u]],
                                      erows_v[b].at[pl.ds(u * 128, 128), :],
                                      sem_e1[b]).wait()

            @pl.loop(0, D_EDGE)
            def _e1sum(d):
                for h in range(CHUNK // L):
                    rows = lanes0 + (d * 2 + h) * L
                    acc = plsc.load_gather(
                        erows_v[b], [rows, jnp.full((L,), 0, jnp.int32)])
                    for k in range(1, S):
                        acc = acc + plsc.load_gather(
                            erows_v[b],
                            [rows, jnp.full((L,), k, jnp.int32)])
                    plsc.store_scatter(
                        e1sum_v,
                        [lanes0 + h * L, jnp.full((L,), 0, jnp.int32) + d],
                        acc)

            for u in range(NG):
                pltpu.make_async_copy(feats_h.at[f2idx_v[b].at[u]],
                                      f2rows_v[b].at[pl.ds(u * GCH, GCH), :],
                                      sem_f2[b]).wait()

            @pl.loop(0, CHUNK)
            def _f2sum(i):
                r0 = i * S
                for c in range(D_IN // L):
                    acc = f2rows_v[b][r0, pl.ds(c * L, L)]
                    for k in range(1, S):
                        acc = acc + f2rows_v[b][r0 + k, pl.ds(c * L, L)]
                    f2sum_v[i, pl.ds(c * L, L)] = acc

            pltpu.sync_copy(e1sum_v, e1s_o.at[mp, j, pl.ds(base, CHUNK), :])
            pltpu.sync_copy(f2sum_v, f2s_o.at[mp, j, pl.ds(base, CHUNK), :])

        front(0, 0)
        mid(0, 0)
        front(1, 1)

        @pl.loop(0, S - 2, step=2)
        def _pipe(j):
            mid(j + 1, 1)
            back(j, 0)
            front(j + 2, 0)
            back(j + 1, 1)
            mid(j + 2, 0)
            front(j + 3, 1)

        mid(S - 1, 1)
        back(S - 2, 0)
        back(S - 1, 1)


_sc_gather = pl.kernel(
    _sc_body,
    out_type=[
        jax.ShapeDtypeStruct((B, D_IN), jnp.float32),
        jax.ShapeDtypeStruct((2, S, B, D_IN), jnp.float32),
        jax.ShapeDtypeStruct((2, S, B, D_EDGE), jnp.float32),
        jax.ShapeDtypeStruct((2, S, B, D_EDGE), jnp.float32),
        jax.ShapeDtypeStruct((2, S, B, D_IN), jnp.float32),
    ],
    mesh=plsc.VectorSubcoreMesh(core_axis_name="c", subcore_axis_name="s",
                                num_cores=NC, num_subcores=NS),
    compiler_params=pltpu.CompilerParams(needs_layout_passes=False,
                                         use_tc_tiling_on_sc=False),
    scratch_types=[
        pltpu.VMEM((CHUNK,), jnp.int32),
        pltpu.VMEM((CHUNK,), jnp.int32),
        pltpu.VMEM((CHUNK, MAX_DEG), jnp.int32),
        pltpu.VMEM((S, CHUNK), jnp.int32),
        pltpu.VMEM((CHUNK,), jnp.int32),
        pltpu.VMEM((CHUNK,), jnp.int32),
        pltpu.VMEM((CHUNK,), jnp.int32),
        pltpu.VMEM((CHUNK,), jnp.int32),
        pltpu.VMEM((CHUNK,), jnp.int32),
        pltpu.VMEM((CHUNK, MAX_DEG), jnp.int32),
        pltpu.VMEM((CHUNK, MAX_DEG), jnp.int32),
        pltpu.VMEM((NG, GCH), jnp.int32),
        pltpu.VMEM((NG, GCH), jnp.int32),
        pltpu.VMEM((4, 128), jnp.int32),
        pltpu.VMEM((4, 128), jnp.int32),
        pltpu.VMEM((512, D_EDGE), jnp.float32),
        pltpu.VMEM((512, D_EDGE), jnp.float32),
        pltpu.VMEM((CHUNK, D_IN), jnp.float32),
        pltpu.VMEM((CHUNK, D_IN), jnp.float32),
        pltpu.VMEM((E2, D_IN), jnp.float32),
        pltpu.VMEM((E2, D_IN), jnp.float32),
        pltpu.VMEM((S, CHUNK, D_EDGE), jnp.float32),
        pltpu.VMEM((CHUNK, D_EDGE), jnp.float32),
        pltpu.VMEM((CHUNK, D_IN), jnp.float32),
        pltpu.VMEM((E2,), jnp.int32),
        pltpu.VMEM((E2,), jnp.int32),
        pltpu.SemaphoreType.DMA,
        pltpu.SemaphoreType.DMA,
        pltpu.SemaphoreType.DMA,
        pltpu.SemaphoreType.DMA,
        pltpu.SemaphoreType.DMA,
        pltpu.SemaphoreType.DMA,
        pltpu.SemaphoreType.DMA,
        pltpu.SemaphoreType.DMA,
        pltpu.SemaphoreType.DMA,
    ],
)


def _tr_body(x_ref, o_ref):
    x = x_ref[0]
    o_ref[0] = x.T


def _make_transpose(dtype, minor, ec):
    return pl.pallas_call(
        _tr_body,
        grid=(2, minor // ec),
        in_specs=[pl.BlockSpec((1, 16, ec), lambda mp, t: (mp, 0, t))],
        out_specs=pl.BlockSpec((1, ec, 16), lambda mp, t: (mp, t, 0)),
        out_shape=jax.ShapeDtypeStruct((2, minor, 16), dtype),
        compiler_params=pltpu.CompilerParams(
            dimension_semantics=("parallel", "parallel"),
        ),
    )


_tr_adj = _make_transpose(jnp.int32, N_NODES, N_NODES)


def _relu(x):
    return jnp.maximum(x, 0.0)


def _mm(a, b):
    return jnp.dot(a, b, preferred_element_type=jnp.float32)


def _tc_body(f0_r, f1_r, e0_r, e1s_r, f2s_r,
             ws0_r, wn0_r, ws1_r, wn1_r, we_r, wfc_r, bfc_r,
             out_r, acc_r, f1s_v, e0s_v, g1s_v, e0ps_v):
    mp = pl.program_id(0)
    inv_s = 1.0 / S

    ws0 = ws0_r[0]
    wn0a = wn0_r[0, :D_IN, :]
    wn0b = wn0_r[0, D_IN:, :]
    ws1 = ws1_r[0]
    wn1a = wn1_r[0, :256, :]
    wn1b = wn1_r[0, 256:, :]
    wea = we_r[0, :256, :]
    web = we_r[0, 256:512, :]
    wec = we_r[0, 512:, :]

    f1s_v[...] = jnp.zeros_like(f1s_v)
    e0s_v[...] = jnp.zeros_like(e0s_v)

    def body1(j, _):
        f1s_v[...] += f1_r[0, j]
        e0s_v[...] += e0_r[0, j]
        return 0

    lax.fori_loop(0, S, body1, 0)

    f0 = f0_r[...]
    g0 = _relu(jnp.concatenate(
        [_mm(f0, ws0),
         _mm(f1s_v[...] * inv_s, wn0a) + _mm(e0s_v[...] * inv_s, wn0b)],
        axis=-1))
    g0ea = _mm(g0, wea)

    g1s_v[...] = jnp.zeros_like(g1s_v)
    e0ps_v[...] = jnp.zeros_like(e0ps_v)

    def body2(j, _):
        g1_j = _relu(jnp.concatenate(
            [_mm(f1_r[0, j], ws0),
             _mm(f2s_r[0, j] * inv_s, wn0a) + _mm(e1s_r[0, j] * inv_s, wn0b)],
            axis=-1))
        e0p_j = _relu(g0ea + _mm(g1_j, web) + _mm(e0_r[0, j], wec))
        g1s_v[...] += g1_j
        e0ps_v[...] += e0p_j
        return 0

    lax.fori_loop(0, S, body2, 0)

    out_mp = _relu(jnp.concatenate(
        [_mm(g0, ws1),
         _mm(g1s_v[...] * inv_s, wn1a) + _mm(e0ps_v[...] * inv_s, wn1b)],
        axis=-1))

    @pl.when(mp == 0)
    def _():
        acc_r[...] = out_mp

    @pl.when(mp == 1)
    def _():
        o = (acc_r[...] + out_mp) * 0.5
        nrm = jnp.sqrt(jnp.sum(o * o, axis=1, keepdims=True))
        o = o / jnp.maximum(nrm, 1e-12)
        out_r[...] = _mm(o, wfc_r[...]) + bfc_r[...]


def _tc_dense(f0, f1, e0, e1s, f2s, Wself0, Wneigh0, Wself1, Wneigh1,
              Wedge0, Wfc, bfc2):
    grid = (2,)
    return pl.pallas_call(
        _tc_body,
        grid=grid,
        in_specs=[
            pl.BlockSpec((B, D_IN), lambda mp: (0, 0)),
            pl.BlockSpec((1, S, B, D_IN), lambda mp: (mp, 0, 0, 0)),
            pl.BlockSpec((1, S, B, D_EDGE), lambda mp: (mp, 0, 0, 0)),
            pl.BlockSpec((1, S, B, D_EDGE), lambda mp: (mp, 0, 0, 0)),
            pl.BlockSpec((1, S, B, D_IN), lambda mp: (mp, 0, 0, 0)),
            pl.BlockSpec((1, D_IN, 128), lambda mp: (mp, 0, 0)),
            pl.BlockSpec((1, D_IN + D_EDGE, 128), lambda mp: (mp, 0, 0)),
            pl.BlockSpec((1, 256, 128), lambda mp: (mp, 0, 0)),
            pl.BlockSpec((1, 256 + D_EDGE, 128), lambda mp: (mp, 0, 0)),
            pl.BlockSpec((1, 512 + D_EDGE, D_EDGE), lambda mp: (mp, 0, 0)),
            pl.BlockSpec((256, 16), lambda mp: (0, 0)),
            pl.BlockSpec((1, 16), lambda mp: (0, 0)),
        ],
        out_specs=pl.BlockSpec((B, 16), lambda mp: (0, 0)),
        out_shape=jax.ShapeDtypeStruct((B, 16), jnp.float32),
        scratch_shapes=[
            pltpu.VMEM((B, 256), jnp.float32),
            pltpu.VMEM((B, D_IN), jnp.float32),
            pltpu.VMEM((B, D_EDGE), jnp.float32),
            pltpu.VMEM((B, 256), jnp.float32),
            pltpu.VMEM((B, D_EDGE), jnp.float32),
        ],
        compiler_params=pltpu.CompilerParams(
            dimension_semantics=("arbitrary",),
        ),
    )(f0, f1, e0, e1s, f2s, Wself0, Wneigh0, Wself1, Wneigh1, Wedge0,
      Wfc, bfc2)


def kernel(ids, feats, adjs, edge_emb, Wself0, Wneigh0, Wself1, Wneigh1,
           Wedge0, Wfc, bfc):
    eeb = jnp.transpose(edge_emb, (0, 2, 1)).reshape(2, 2, 8, 6250, 128)
    eeb = jnp.transpose(eeb, (0, 1, 3, 2, 4)).reshape(
        2 * MAX_DEG * N_NODES, D_EDGE)
    ad_c = _tr_adj(jnp.transpose(adjs, (0, 2, 1)))
    ad_flat = ad_c.reshape(2 * N_NODES, MAX_DEG)
    f0, f1, e0, e1s, f2s = _sc_gather(ids, feats, ad_flat, eeb)
    return _tc_dense(f0, f1, e0, e1s, f2s, Wself0, Wneigh0, Wself1,
                     Wneigh1, Wedge0, Wfc, bfc.reshape(1, 16))
